# Initial kernel scaffold; baseline (speedup 1.0000x reference)
import jax
import jax.numpy as jnp
from jax.experimental import pallas as pl


def kernel(x_atm, x_bnd, x_ang, mask_dih_ang, edge_index_G, edge_index_L, x_atm_batch, x_bnd_batch, x_ang_batch, params):
    raise NotImplementedError("write your pallas kernel here")


# speedup vs baseline: 1.6335x; 1.6335x over previous
import functools

import jax
import jax.numpy as jnp
from jax import lax
from jax.experimental import pallas as pl
from jax.experimental.pallas import tpu as pltpu
from jax.experimental.pallas import tpu_sc as plsc

F32 = jnp.float32
I32 = jnp.int32

D = 128
CUTOFF = 3.0
EPS = 1e-5
NGR = 64

NC, NS = 2, 16
NW = NC * NS
TPB = 256
PADU = NW * TPB
BN = 512


def _gauss(x, mu, step):
    diff = (x[:, :, None] - mu) / step
    return jnp.exp(-(diff ** 2)) / 1.12


def _embed_bnd_body(x_ref, o_ref):
    x = x_ref[...][:, :, None] + 1e-5
    n = lax.broadcasted_iota(I32, (1, 1, D), 2).astype(F32) + 1.0
    arg = n * jnp.pi * x / CUTOFF
    o_ref[...] = ((2.0 / CUTOFF) ** 0.5) * jnp.sin(arg) / x


def _embed_ang_body(x_ref, m_ref, mu64_ref, mu32_ref, st_ref, o_ref):
    x = x_ref[...]
    m = m_ref[...][:, :, None] > 0.5
    mu64 = mu64_ref[...][None]
    mu32 = mu32_ref[...][None]
    s64 = st_ref[0, 0]
    s32 = st_ref[0, 1]
    ca = jnp.cos(x)
    sa = jnp.sin(x)
    h_plain = _gauss(ca, mu64, s64)
    h_dih = jnp.concatenate([_gauss(ca, mu32, s32), _gauss(sa, mu32, s32)],
                            axis=-1)
    first = jnp.where(m, 0.0, h_plain)
    second = jnp.where(m, h_dih, 0.0)
    o_ref[...] = jnp.concatenate([first, second], axis=-1)


def _embed_atm_body(i_ref, e_ref, o_ref):
    ids = i_ref[...][:, :, None]
    e0 = e_ref[0:1, :][None]
    e1 = e_ref[1:2, :][None]
    e2 = e_ref[2:3, :][None]
    o_ref[...] = jnp.where(ids == 0, e0, jnp.where(ids == 1, e1, e2))


@functools.lru_cache(maxsize=None)
def _embed_bnd_call(nrow, bx, interpret=False):
    return pl.pallas_call(
        _embed_bnd_body,
        grid=(nrow // bx,),
        in_specs=[pl.BlockSpec((bx, D), lambda i: (i, 0))],
        out_specs=pl.BlockSpec((bx, D, D), lambda i: (i, 0, 0)),
        out_shape=jax.ShapeDtypeStruct((nrow, D, D), F32),
        interpret=interpret)


@functools.lru_cache(maxsize=None)
def _embed_ang_call(nrow, bx, interpret=False):
    return pl.pallas_call(
        _embed_ang_body,
        grid=(nrow // bx,),
        in_specs=[pl.BlockSpec((bx, D), lambda i: (i, 0)),
                  pl.BlockSpec((bx, D), lambda i: (i, 0)),
                  pl.BlockSpec((1, D // 2), lambda i: (0, 0)),
                  pl.BlockSpec((1, D // 4), lambda i: (0, 0)),
                  pl.BlockSpec((1, 8), lambda i: (0, 0))],
        out_specs=pl.BlockSpec((bx, D, D), lambda i: (i, 0, 0)),
        out_shape=jax.ShapeDtypeStruct((nrow, D, D), F32),
        interpret=interpret)


@functools.lru_cache(maxsize=None)
def _embed_atm_call(nrow, bx, interpret=False):
    return pl.pallas_call(
        _embed_atm_body,
        grid=(nrow // bx,),
        in_specs=[pl.BlockSpec((bx, D), lambda i: (i, 0)),
                  pl.BlockSpec((8, D), lambda i: (0, 0))],
        out_specs=pl.BlockSpec((bx, D, D), lambda i: (i, 0, 0)),
        out_shape=jax.ShapeDtypeStruct((nrow, D, D), F32),
        interpret=interpret)


def _node_body(x_ref, w_ref, b_ref, a_ref, bd_ref, s_ref):
    y = jnp.dot(x_ref[...], w_ref[...], preferred_element_type=F32) + b_ref[...]
    a_ref[...] = y[:, :D]
    bd_ref[...] = y[:, D:3 * D]
    s_ref[...] = y[:, 3 * D:]


@functools.lru_cache(maxsize=None)
def _node_call(n, interpret=False):
    return pl.pallas_call(
        _node_body,
        grid=(n // BN,),
        in_specs=[pl.BlockSpec((BN, D), lambda i: (i, 0)),
                  pl.BlockSpec((D, 4 * D), lambda i: (0, 0)),
                  pl.BlockSpec((1, 4 * D), lambda i: (0, 0))],
        out_specs=[pl.BlockSpec((BN, D), lambda i: (i, 0)),
                   pl.BlockSpec((BN, 2 * D), lambda i: (i, 0)),
                   pl.BlockSpec((BN, D), lambda i: (i, 0))],
        out_shape=(jax.ShapeDtypeStruct((n, D), F32),
                   jax.ShapeDtypeStruct((n, 2 * D), F32),
                   jax.ShapeDtypeStruct((n, D), F32)),
        interpret=interpret)


def _edge_body(ai_ref, bdj_ref, ang_ref, we_ref, bs_ref, ms_ref, eo_ref):
    ang = ang_ref[...]
    bdj = bdj_ref[...]
    eij = ((ai_ref[...] + bdj[:, :D])
           + (jnp.dot(ang, we_ref[...], preferred_element_type=F32)
              + bs_ref[...]))
    sig = jax.nn.sigmoid(eij)
    msg = sig * bdj[:, D:]
    ms_ref[...] = jnp.concatenate([msg, sig], axis=1)
    eo_ref[...] = ang + eij * sig


@functools.lru_cache(maxsize=None)
def _edge_call(e, interpret=False):
    return pl.pallas_call(
        _edge_body,
        grid=(e // BN,),
        in_specs=[pl.BlockSpec((BN, D), lambda i: (i, 0)),
                  pl.BlockSpec((BN, 2 * D), lambda i: (i, 0)),
                  pl.BlockSpec((BN, D), lambda i: (i, 0)),
                  pl.BlockSpec((D, D), lambda i: (0, 0)),
                  pl.BlockSpec((1, D), lambda i: (0, 0))],
        out_specs=[pl.BlockSpec((BN, 2 * D), lambda i: (i, 0)),
                   pl.BlockSpec((BN, D), lambda i: (i, 0))],
        out_shape=(jax.ShapeDtypeStruct((e, 2 * D), F32),
                   jax.ShapeDtypeStruct((e, D), F32)),
        interpret=interpret)


def _comb_body(x_ref, s_ref, nd_ref, o_ref):
    nd = nd_ref[...]
    t = s_ref[...] + nd[:, :D] / (nd[:, D:] + EPS)
    o_ref[...] = x_ref[...] + t * jax.nn.sigmoid(t)


@functools.lru_cache(maxsize=None)
def _comb_call(n, npad, interpret=False):
    return pl.pallas_call(
        _comb_body,
        grid=(n // BN,),
        in_specs=[pl.BlockSpec((BN, D), lambda i: (i, 0)),
                  pl.BlockSpec((BN, D), lambda i: (i, 0)),
                  pl.BlockSpec((BN, 2 * D), lambda i: (i, 0))],
        out_specs=pl.BlockSpec((BN, D), lambda i: (i, 0)),
        out_shape=jax.ShapeDtypeStruct((n, D), F32),
        interpret=interpret)


def _head_body(h_ref, w_ref, b_ref, g_ref, hs_ref, pool_ref):
    t = jnp.dot(h_ref[...], w_ref[...], preferred_element_type=F32) + b_ref[...]
    s = jnp.maximum(t, 0.0) + jnp.log1p(jnp.exp(-jnp.abs(t)))
    hs_ref[...] = s
    seg = g_ref[...]
    segb = lax.broadcast_in_dim(seg, (NGR, BN), (1,))
    oh = (lax.broadcasted_iota(I32, (NGR, BN), 0) == segb).astype(F32)
    part = jnp.dot(oh, s, preferred_element_type=F32)

    @pl.when(pl.program_id(0) == 0)
    def _():
        pool_ref[...] = jnp.zeros_like(pool_ref)

    pool_ref[...] += part


@functools.lru_cache(maxsize=None)
def _head_call(n, interpret=False):
    return pl.pallas_call(
        _head_body,
        grid=(n // BN,),
        in_specs=[pl.BlockSpec((BN, D), lambda i: (i, 0)),
                  pl.BlockSpec((D, 8), lambda i: (0, 0)),
                  pl.BlockSpec((1, 8), lambda i: (0, 0)),
                  pl.BlockSpec((BN,), lambda i: (i,))],
        out_specs=[pl.BlockSpec((BN, 8), lambda i: (i, 0)),
                   pl.BlockSpec((NGR, 8), lambda i: (0, 0))],
        out_shape=(jax.ShapeDtypeStruct((n, 8), F32),
                   jax.ShapeDtypeStruct((NGR, 8), F32)),
        interpret=interpret)


def _mesh():
    return plsc.VectorSubcoreMesh(core_axis_name="c", subcore_axis_name="s")


@functools.lru_cache(maxsize=None)
def _gather2_call(na, e):
    nblk = e // 128
    npw = (nblk + NW - 1) // NW

    @functools.partial(
        pl.kernel,
        out_type=(jax.ShapeDtypeStruct((e, D), F32),
                  jax.ShapeDtypeStruct((e, 2 * D), F32)),
        mesh=_mesh(),
        scratch_types=[pltpu.VMEM((128,), I32),
                       pltpu.VMEM((128,), I32),
                       pltpu.VMEM((128, D), F32),
                       pltpu.VMEM((128, 2 * D), F32),
                       pltpu.SemaphoreType.DMA,
                       pltpu.SemaphoreType.DMA])
    def k(a_hbm, bd_hbm, ii_hbm, jj_hbm, ai_hbm, bdj_hbm,
          iv, jv, ra, rb, s1, s2):
        wid = lax.axis_index("s") * NC + lax.axis_index("c")

        def body(kk, carry):
            b = wid + NW * kk

            @pl.when(b < nblk)
            def _():
                base = b * 128
                pltpu.sync_copy(ii_hbm.at[pl.ds(base, 128)], iv)
                pltpu.sync_copy(jj_hbm.at[pl.ds(base, 128)], jv)
                ca = pltpu.async_copy(a_hbm.at[iv], ra, s1)
                cb = pltpu.async_copy(bd_hbm.at[jv], rb, s2)
                ca.wait()
                cb.wait()
                pltpu.sync_copy(ra, ai_hbm.at[pl.ds(base, 128)])
                pltpu.sync_copy(rb, bdj_hbm.at[pl.ds(base, 128)])

            return carry

        lax.fori_loop(0, npw, body, 0)

    return k


def _svmem(ref, idx):
    return ref[pl.ds(idx, 16)][0]


@functools.lru_cache(maxsize=None)
def _scatter_call(e, npad):
    nphase = npad // PADU

    @functools.partial(
        pl.kernel,
        out_type=jax.ShapeDtypeStruct((npad, 2 * D), F32),
        mesh=_mesh(),
        scratch_types=[pltpu.VMEM((768,), I32),
                       pltpu.VMEM((128,), I32),
                       pltpu.VMEM((128,), I32),
                       pltpu.VMEM((128,), I32),
                       pltpu.VMEM((128, 2 * D), F32),
                       pltpu.VMEM((TPB + 8, 2 * D), F32),
                       pltpu.SemaphoreType.DMA])
    def k(ms_hbm, perm_hbm, dst_hbm, cs_hbm, zero_hbm, nd_hbm,
          cs_v, pidx, dstv, sidx, rows, accum, sem):
        wid = lax.axis_index("s") * NC + lax.axis_index("c")
        pltpu.sync_copy(cs_hbm, cs_v)

        def phase_body(p, carry):
            rowbase = p * PADU + wid * TPB
            fi = p * NW + wid
            start = _svmem(cs_v, fi)
            end = _svmem(cs_v, fi + 1)
            pltpu.sync_copy(zero_hbm, accum)

            blo = lax.shift_right_logical(start, 7)
            bhi = lax.shift_right_logical(end + 127, 7)

            def blk(b, c2):
                base = b * 128
                pltpu.sync_copy(perm_hbm.at[pl.ds(base, 128)], pidx)
                pltpu.sync_copy(dst_hbm.at[pl.ds(base, 128)], dstv)
                pltpu.async_copy(ms_hbm.at[pidx], rows, sem).wait()
                for g in range(8):
                    v = dstv[pl.ds(g * 16, 16)]
                    rel = v - rowbase
                    ok = (rel >= 0) & (rel < TPB)
                    sidx[pl.ds(g * 16, 16)] = jnp.where(ok, rel, TPB)

                def q_body(q, c3):
                    sv = sidx[pl.ds(q * 16, 16)]
                    for l in range(16):
                        r = sv[l]
                        e = q * 16 + l
                        for g in range(16):
                            v = rows[e, pl.ds(g * 16, 16)]
                            plsc.addupdate(accum.at[r, pl.ds(g * 16, 16)], v)
                    return c3

                lax.fori_loop(0, 8, q_body, 0)
                return c2

            lax.fori_loop(blo, bhi, blk, 0)
            pltpu.sync_copy(accum.at[pl.ds(0, TPB)],
                            nd_hbm.at[pl.ds(rowbase, TPB)])
            return carry

        lax.fori_loop(0, nphase, phase_body, 0)

    return k


def _pack_egconv(p):
    w4 = jnp.concatenate([p['W_src']['w'], p['W_dst']['w'],
                          p['W_d']['w'], p['W_s']['w']], axis=1)
    b4 = jnp.concatenate([p['W_src']['b'], p['W_dst']['b'],
                          p['W_d']['b'], p['W_s']['b']])[None, :]
    we = p['W_e']['w']
    be = p['W_e']['b'][None, :]
    return w4, b4, we, be


def _sort_meta(dst, n):
    e = dst.shape[0]
    npad = ((n + PADU - 1) // PADU) * PADU
    nwin = npad // TPB
    dst_s, perm = lax.sort_key_val(dst, jnp.arange(e, dtype=I32))
    bounds = (jnp.arange(nwin + 1, dtype=I32) * TPB).astype(dst_s.dtype)
    cs = jnp.searchsorted(dst_s, bounds, side='left').astype(I32)
    cs = jnp.full((768,), e, dtype=I32).at[:nwin + 1].set(cs)
    return perm, dst_s.astype(I32), cs, npad


def _egconv(x, n, ang, e, ii, jj, perm, dst_s, cs, npad, wpack, zeros128):
    w4, b4, we, bsum = wpack
    a, bd, s = _node_call(n)(x, w4, b4)
    ai, bdj = _gather2_call(n, e)(a, bd, ii, jj)
    ms, ang_out = _edge_call(e)(ai, bdj, ang, we, bsum)
    nd = _scatter_call(e, npad)(ms, perm, dst_s, cs, zeros128)
    x_out = _comb_call(n, npad)(x, s, nd)
    return x_out, ang_out


def _pad_to(x, n, val):
    return jnp.concatenate(
        [x, jnp.full((n - x.shape[0],) + x.shape[1:], val, x.dtype)])


def kernel(x_atm, x_bnd, x_ang, mask_dih_ang, edge_index_G, edge_index_L,
           x_atm_batch, x_bnd_batch, x_ang_batch, params):
    n_atm = x_atm.shape[0]
    n_bnd = x_bnd.shape[0]
    n_ang = x_ang.shape[0]
    na_pad = ((n_atm + PADU - 1) // PADU) * PADU
    nb_pad = ((n_bnd + PADU - 1) // PADU) * PADU
    ng_pad = ((n_ang + PADU - 1) // PADU) * PADU

    iG = _pad_to(edge_index_G[0].astype(I32), nb_pad, na_pad)
    jG = _pad_to(edge_index_G[1].astype(I32), nb_pad, 0)
    iL = _pad_to(edge_index_L[0].astype(I32), ng_pad, nb_pad)
    jL = _pad_to(edge_index_L[1].astype(I32), ng_pad, 0)

    permL, dstL, csL, npadL = _sort_meta(iL, n_bnd)
    permG, dstG, csG, npadG = _sort_meta(iG, n_atm)
    zeros128 = jnp.zeros((TPB + 8, 2 * D), F32)

    ids_pad = _pad_to(x_atm.astype(I32), na_pad, 0)
    emb_pad = jnp.concatenate(
        [params['embed_atm'], jnp.zeros((8 - 3, D), F32)], axis=0)

    h_atm = _embed_atm_call(na_pad // D, 8)(
        ids_pad.reshape(na_pad // D, D), emb_pad).reshape(na_pad, D)
    h_bnd = _embed_bnd_call(nb_pad // D, 8)(
        _pad_to(x_bnd, nb_pad, 0.0).reshape(nb_pad // D, D)).reshape(nb_pad, D)
    mu64 = jnp.linspace(-1.0, 1.0, D // 2, dtype=F32)[None, :]
    mu32 = jnp.linspace(-1.0, 1.0, D // 4, dtype=F32)[None, :]
    steps = jnp.zeros((1, 8), F32)
    steps = steps.at[0, 0].set(mu64[0, 1] - mu64[0, 0])
    steps = steps.at[0, 1].set(mu32[0, 1] - mu32[0, 0])
    h_ang = _embed_ang_call(ng_pad // D, 8)(
        _pad_to(x_ang, ng_pad, 0.0).reshape(ng_pad // D, D),
        _pad_to(mask_dih_ang.astype(F32), ng_pad, 0.0).reshape(
            ng_pad // D, D),
        mu64, mu32, steps).reshape(ng_pad, D)

    packsL = [_pack_egconv(p) for p in params['bnd_ang']]
    packsG = [_pack_egconv(p) for p in params['atm_bnd']]

    for t in range(len(packsL)):
        h_bnd, h_ang = _egconv(h_bnd, nb_pad, h_ang, ng_pad, iL, jL,
                               permL, dstL, csL, npadL, packsL[t], zeros128)
        h_atm, h_bnd = _egconv(h_atm, na_pad, h_bnd, nb_pad, iG, jG,
                               permG, dstG, csG, npadG, packsG[t], zeros128)

    def head_pack(p):
        w = jnp.concatenate([p['w'], jnp.zeros((D, 8 - 3), F32)], axis=1)
        b = jnp.concatenate([p['b'], jnp.zeros((8 - 3,), F32)])[None, :]
        return w, b

    wa, ba = head_pack(params['head_atm'])
    wb, bb = head_pack(params['head_bnd'])
    wc, bc = head_pack(params['head_ang'])

    hsa, pa = _head_call(na_pad)(
        h_atm, wa, ba, _pad_to(x_atm_batch.astype(I32), na_pad, NGR))
    hsb, pb = _head_call(nb_pad)(
        h_bnd, wb, bb, _pad_to(x_bnd_batch.astype(I32), nb_pad, NGR))
    hsc, pc = _head_call(ng_pad)(
        h_ang, wc, bc, _pad_to(x_ang_batch.astype(I32), ng_pad, NGR))

    out = (pa + pb + pc)[:, :3]
    return out, (hsa[:n_atm, :3], hsb[:n_bnd, :3], hsc[:n_ang, :3])
